# baseline (speedup 1.0000x reference)
import jax
import jax.numpy as jnp
from jax.experimental import pallas as pl

NEG_SLOPE = 0.2


def _fc_body(p_ref, w_ref, b_ref, o_ref):
    o_ref[...] = jnp.maximum(
        jnp.dot(p_ref[...], w_ref[...], preferred_element_type=jnp.float32)
        + b_ref[...], 0.0)


def _gat_conv(x, edge_index, W, a_src, a_dst, bias, heads, out_ch, concat):
    n = x.shape[0]
    loop = jnp.arange(n, dtype=edge_index.dtype)
    src = jnp.concatenate([edge_index[0], loop])
    dst = jnp.concatenate([edge_index[1], loop])
    h = (x @ W).reshape(n, heads, out_ch)
    alpha_src = (h * a_src[None, :, :]).sum(-1)
    alpha_dst = (h * a_dst[None, :, :]).sum(-1)
    alpha = alpha_src[src] + alpha_dst[dst]
    alpha = jax.nn.leaky_relu(alpha, NEG_SLOPE)
    amax = jax.ops.segment_max(alpha, dst, num_segments=n)
    alpha = jnp.exp(alpha - amax[dst])
    denom = jax.ops.segment_sum(alpha, dst, num_segments=n)
    alpha = alpha / (denom[dst] + 1e-16)
    msg = h[src] * alpha[:, :, None]
    out = jax.ops.segment_sum(msg, dst, num_segments=n)
    if concat:
        out = out.reshape(n, heads * out_ch)
    else:
        out = out.mean(axis=1)
    return out + bias


def kernel(x, edge_index, batch, W1, a_src1, a_dst1, b1, W2, a_src2, a_dst2, b2, Wf, bf):
    si = jnp.argsort(edge_index[1])
    edge_index = edge_index[:, si]
    N, F_IN = x.shape
    HEADS = a_src1.shape[0]
    OUT = W2.shape[1]
    B = 256
    h = _gat_conv(x, edge_index, W1, a_src1, a_dst1, b1, HEADS, F_IN, True)
    h = jax.nn.relu(h)
    h = _gat_conv(h, edge_index, W2, a_src2, a_dst2, b2, 1, OUT, True)
    h = jax.nn.relu(h)
    pooled = jax.ops.segment_max(h, batch, num_segments=B)
    out = pl.pallas_call(
        _fc_body,
        out_shape=jax.ShapeDtypeStruct((B, OUT), jnp.float32),
    )(pooled, Wf, bf.reshape(1, OUT))
    return out


# speedup vs baseline: 14.7078x; 14.7078x over previous
import functools

import jax
import jax.numpy as jnp
from jax import lax
from jax.experimental import pallas as pl
from jax.experimental.pallas import tpu as pltpu
from jax.experimental.pallas import tpu_sc as plsc

f32 = jnp.float32
i32 = jnp.int32

NEG = 0.2
N = 50000
E = 800000
EP = 851968
NT = 50176
HP = 800
OUT = 128
NH = 10

CB1 = 1792
NBLK1 = 28
ACC1 = 1824
CB2 = 12544
ACC2 = 12576

CH_A = EP // 32
GA = 128
NG_A = CH_A // GA
CH_B = EP // 16
SCCH = 512
NCH_B = CH_B // SCCH
SELCAP = 544


@functools.lru_cache(maxsize=1)
def _mesh():
    return plsc.VectorSubcoreMesh(core_axis_name="c", subcore_axis_name="s",
                                  num_cores=2, num_subcores=16)


_SC_PARAMS = dict(use_tc_tiling_on_sc=False, needs_layout_passes=False)
_Z16F = lambda: jnp.zeros((16,), f32)
_Z16I = lambda: jnp.zeros((16,), i32)


def _bcast16(vec, idx_scalar):
    idx = jnp.full((16, 1), idx_scalar, i32)
    dn = lax.GatherDimensionNumbers(offset_dims=(), collapsed_slice_dims=(0,),
                                    start_index_map=(0,))
    return lax.gather(vec, idx, dn, (1,),
                      mode=lax.GatherScatterMode.PROMISE_IN_BOUNDS)


def _ones_zeros(m):
    return jnp.where(m, jnp.full((16,), 1, i32), jnp.full((16,), 0, i32))


def _k1_body(xp_ref, w1_ref, asrc_ref, adst_ref, h_ref, as_ref, ad_ref):
    h = jnp.dot(xp_ref[...], w1_ref[...], preferred_element_type=f32)
    h_ref[...] = h
    as_ref[...] = jnp.dot(h, asrc_ref[...], preferred_element_type=f32)
    ad_ref[...] = jnp.dot(h, adst_ref[...], preferred_element_type=f32)


def _k1(xp, w1p, asrc, adst):
    return pl.pallas_call(
        _k1_body,
        grid=(NT // 128,),
        in_specs=[
            pl.BlockSpec((128, 128), lambda i: (i, 0)),
            pl.BlockSpec((128, HP), lambda i: (0, 0)),
            pl.BlockSpec((HP, 16), lambda i: (0, 0)),
            pl.BlockSpec((HP, 16), lambda i: (0, 0)),
        ],
        out_specs=[
            pl.BlockSpec((128, HP), lambda i: (i, 0)),
            pl.BlockSpec((128, 16), lambda i: (i, 0)),
            pl.BlockSpec((128, 16), lambda i: (i, 0)),
        ],
        out_shape=[
            jax.ShapeDtypeStruct((NT, HP), f32),
            jax.ShapeDtypeStruct((NT, 16), f32),
            jax.ShapeDtypeStruct((NT, 16), f32),
        ],
    )(xp, w1p, asrc, adst)


def _recip_body(a_ref, b_ref, o_ref):
    o_ref[...] = 1.0 / (a_ref[...] + b_ref[...] + 1e-16)


def _recip(a, b, rows_per_block):
    r = a.shape[0]
    return pl.pallas_call(
        _recip_body,
        grid=(r // rows_per_block,),
        in_specs=[pl.BlockSpec((rows_per_block, 128), lambda i: (i, 0))] * 2,
        out_specs=pl.BlockSpec((rows_per_block, 128), lambda i: (i, 0)),
        out_shape=jax.ShapeDtypeStruct((r, 128), f32),
    )(a, b)


def _k3_body(o1_ref, w2_ref, b1p_ref, a2s_ref, a2d_ref, h2_ref, as_ref,
             ad_ref):
    t = jnp.maximum(o1_ref[...] + b1p_ref[...], 0.0)
    h2 = jnp.dot(t, w2_ref[...], preferred_element_type=f32)
    h2_ref[...] = h2
    as_ref[...] = jnp.sum(h2 * a2s_ref[...], axis=1)[None, None, :]
    ad_ref[...] = jnp.sum(h2 * a2d_ref[...], axis=1)[None, None, :]


def _k3(out1, w2p, b1p, a2s, a2d):
    return pl.pallas_call(
        _k3_body,
        grid=(NT // 128,),
        in_specs=[
            pl.BlockSpec((128, HP), lambda i: (i, 0)),
            pl.BlockSpec((HP, OUT), lambda i: (0, 0)),
            pl.BlockSpec((1, HP), lambda i: (0, 0)),
            pl.BlockSpec((1, OUT), lambda i: (0, 0)),
            pl.BlockSpec((1, OUT), lambda i: (0, 0)),
        ],
        out_specs=[
            pl.BlockSpec((128, OUT), lambda i: (i, 0)),
            pl.BlockSpec((1, 1, OUT), lambda i: (i, 0, 0)),
            pl.BlockSpec((1, 1, OUT), lambda i: (i, 0, 0)),
        ],
        out_shape=[
            jax.ShapeDtypeStruct((NT, OUT), f32),
            jax.ShapeDtypeStruct((NT // 128, 1, OUT), f32),
            jax.ShapeDtypeStruct((NT // 128, 1, OUT), f32),
        ],
    )(out1, w2p, b1p, a2s, a2d)


def _k6_body(p_ref, wf_ref, bf_ref, o_ref):
    m = jnp.max(p_ref[...], axis=0)
    o_ref[...] = jnp.maximum(
        jnp.dot(m, wf_ref[...], preferred_element_type=f32) + bf_ref[...],
        0.0)


def _k6(partials, wf, bfr):
    return pl.pallas_call(
        _k6_body,
        out_shape=jax.ShapeDtypeStruct((256, OUT), f32),
    )(partials, wf, bfr)


def _k2a_body(src_hbm, dst_hbm, as1_hbm, ad1_hbm, p_hbm, parts_hbm,
              srcv, dstv, asr, adr, pr, zb, dn_sh, sem):
    cid = lax.axis_index("c")
    sid = lax.axis_index("s")
    wid = sid * 2 + cid

    def zf(i, _):
        zb[i, :] = _Z16F()
        return 0
    lax.fori_loop(0, 392, zf, 0)

    def zdma(i, _):
        pltpu.sync_copy(zb, dn_sh.at[pl.ds(sid * 3136 + i * 392, 392)])
        return 0
    lax.fori_loop(0, 8, zdma, 0)
    plsc.subcore_barrier()

    base0 = wid * CH_A

    def body(g, _):
        b = base0 + g * GA
        pltpu.sync_copy(src_hbm.at[pl.ds(b, GA)], srcv)
        pltpu.sync_copy(dst_hbm.at[pl.ds(b, GA)], dstv)
        pltpu.async_copy(as1_hbm.at[srcv], asr, sem).wait()
        pltpu.async_copy(ad1_hbm.at[dstv], adr, sem).wait()

        def pe(j, _):
            s = asr[j, :] + adr[j, :]
            pr[j, :] = jnp.exp(jnp.maximum(s, s * NEG))
            return 0
        lax.fori_loop(0, GA, pe, 0)
        pltpu.sync_copy(pr, dn_sh.at[dstv], add=True)
        pltpu.sync_copy(pr, p_hbm.at[pl.ds(b, GA)])
        return 0
    lax.fori_loop(0, NG_A, body, 0)
    plsc.subcore_barrier()
    pltpu.sync_copy(dn_sh.at[pl.ds(sid * 3136, 3136)],
                    parts_hbm.at[cid].at[pl.ds(sid * 3136, 3136)])


def _k2a(src, dst, as1, ad1):
    k = pl.kernel(
        _k2a_body,
        out_type=[
            jax.ShapeDtypeStruct((EP, 16), f32),
            jax.ShapeDtypeStruct((2, NT, 16), f32),
        ],
        mesh=_mesh(),
        compiler_params=pltpu.CompilerParams(**_SC_PARAMS),
        scratch_types=[
            pltpu.VMEM((GA,), i32),
            pltpu.VMEM((GA,), i32),
            pltpu.VMEM((GA, 16), f32),
            pltpu.VMEM((GA, 16), f32),
            pltpu.VMEM((GA, 16), f32),
            pltpu.VMEM((392, 16), f32),
            pltpu.VMEM_SHARED((NT, 16), f32),
            pltpu.SemaphoreType.DMA,
        ],
    )
    return k(src, dst, as1, ad1)


def _k4a_body(src_hbm, dst_hbm, as2_hbm, ad2_hbm, p_hbm, parts_hbm,
              srcv, dstv, asv, adv, pv, zb, dn_sh, sem):
    cid = lax.axis_index("c")
    sid = lax.axis_index("s")
    wid = sid * 2 + cid

    def zf(i, _):
        zb[pl.ds(i * 16, 16)] = _Z16F()
        return 0
    lax.fori_loop(0, 392 // 16 + 1, zf, 0)

    def zdma(i, _):
        pltpu.sync_copy(zb.at[pl.ds(0, 392)],
                        dn_sh.at[pl.ds(sid * 3136 + i * 392, 392)])
        return 0
    lax.fori_loop(0, 8, zdma, 0)
    plsc.subcore_barrier()

    base0 = wid * CH_A

    def body(g, _):
        b = base0 + g * GA
        pltpu.sync_copy(src_hbm.at[pl.ds(b, GA)], srcv)
        pltpu.sync_copy(dst_hbm.at[pl.ds(b, GA)], dstv)
        pltpu.async_copy(as2_hbm.at[srcv], asv, sem).wait()
        pltpu.async_copy(ad2_hbm.at[dstv], adv, sem).wait()
        for q in range(GA // 16):
            s = asv[pl.ds(q * 16, 16)] + adv[pl.ds(q * 16, 16)]
            pv[pl.ds(q * 16, 16)] = jnp.exp(jnp.maximum(s, s * NEG))
        pltpu.sync_copy(pv, dn_sh.at[dstv], add=True)
        pltpu.sync_copy(pv, p_hbm.at[pl.ds(b, GA)])
        return 0
    lax.fori_loop(0, NG_A, body, 0)
    plsc.subcore_barrier()
    pltpu.sync_copy(dn_sh.at[pl.ds(sid * 3136, 3136)],
                    parts_hbm.at[cid].at[pl.ds(sid * 3136, 3136)])


def _k4a(src, dst, as2, ad2):
    k = pl.kernel(
        _k4a_body,
        out_type=[
            jax.ShapeDtypeStruct((EP,), f32),
            jax.ShapeDtypeStruct((2, NT), f32),
        ],
        mesh=_mesh(),
        compiler_params=pltpu.CompilerParams(**_SC_PARAMS),
        scratch_types=[
            pltpu.VMEM((GA,), i32),
            pltpu.VMEM((GA,), i32),
            pltpu.VMEM((GA,), f32),
            pltpu.VMEM((GA,), f32),
            pltpu.VMEM((GA,), f32),
            pltpu.VMEM((400,), f32),
            pltpu.VMEM_SHARED((NT,), f32),
            pltpu.SemaphoreType.DMA,
        ],
    )
    return k(src, dst, as2, ad2)


def _k2c_body(src_hbm, dst_hbm, p_hbm, h1p_hbm, rip_hbm, out1_hbm,
              dstc, srcc, seld, sels, sele, gidx, pbuf, hbuf, wscr,
              ripf, zbuf, acc_sh, sem, sem2):
    cid = lax.axis_index("c")
    sid = lax.axis_index("s")
    lanes = lax.iota(i32, 16)

    def zf(i, _):
        for j in range(HP // 16):
            zbuf[i, pl.ds(j * 16, 16)] = _Z16F()
        return 0
    lax.fori_loop(0, 6, zf, 0)

    def sz(i, _):
        seld[pl.ds(i * 16, 16)] = _Z16I()
        sels[pl.ds(i * 16, 16)] = _Z16I()
        sele[pl.ds(i * 16, 16)] = _Z16I()
        return 0
    lax.fori_loop(0, SELCAP // 16, sz, 0)

    scan_base = sid * CH_B

    def process_group(goff, nval):
        dl = seld[pl.ds(goff, 16)]
        sl = sels[pl.ds(goff, 16)]
        ei = sele[pl.ds(goff, 16)]
        gidx[...] = jnp.where(lanes < nval, dl, jnp.full((16,), CB1, i32))
        c1 = pltpu.async_copy(p_hbm.at[ei], pbuf, sem)
        c2 = pltpu.async_copy(h1p_hbm.at[sl], hbuf, sem2)
        c1.wait()
        c2.wait()

        def pe(j, _):
            wv = pbuf[j, :]
            for h in range(NH):
                wh = _bcast16(wv, h)
                for q in range(5):
                    col = (h * 5 + q) * 16
                    hbuf[j, pl.ds(col, 16)] = hbuf[j, pl.ds(col, 16)] * wh
            return 0
        lax.fori_loop(0, nval, pe, 0)
        pltpu.sync_copy(hbuf, acc_sh.at[gidx], add=True)

    def block_loop(blk2, _):
        blk = 2 * blk2 + cid
        lo = blk * CB1
        hi = jnp.minimum(lo + CB1, jnp.int32(N))

        def zd(i, _):
            pltpu.sync_copy(zbuf, acc_sh.at[pl.ds(sid * 114 + i * 6, 6)])
            return 0
        lax.fori_loop(0, 19, zd, 0)
        plsc.subcore_barrier()

        def chunk_loop(c, cnt):
            cb = scan_base + c * SCCH
            pltpu.sync_copy(dst_hbm.at[pl.ds(cb, SCCH)], dstc)
            pltpu.sync_copy(src_hbm.at[pl.ds(cb, SCCH)], srcc)

            def vec_loop(v, cnt):
                dv = dstc[pl.ds(v * 16, 16)]
                sv = srcc[pl.ds(v * 16, 16)]
                m = (dv >= lo) & (dv < hi)
                cs = plsc.cumsum(_ones_zeros(m))
                pos = jnp.where(m, cnt + cs - 1,
                                jnp.full((16,), 528, i32) + lanes)
                plsc.store_scatter(seld, [pos], dv - lo)
                plsc.store_scatter(sels, [pos], sv)
                ev = jnp.full((16,), cb + v * 16, i32) + lanes
                plsc.store_scatter(sele, [pos], ev)
                return cnt + cs[15]
            cnt = lax.fori_loop(0, 32, vec_loop, cnt)
            ng = cnt // 16

            def grp(g, _):
                process_group(g * 16, jnp.int32(16))
                return 0
            lax.fori_loop(0, ng, grp, 0)
            rem = cnt - ng * 16
            td = seld[pl.ds(ng * 16, 16)]
            ts = sels[pl.ds(ng * 16, 16)]
            te = sele[pl.ds(ng * 16, 16)]
            seld[pl.ds(0, 16)] = td
            sels[pl.ds(0, 16)] = ts
            sele[pl.ds(0, 16)] = te
            return rem
        cnt_end = lax.fori_loop(0, NCH_B, chunk_loop, jnp.int32(0))

        @pl.when(cnt_end > 0)
        def _():
            process_group(0, cnt_end)
        plsc.subcore_barrier()

        pltpu.sync_copy(rip_hbm.at[pl.ds(lo + sid * 112, 112)], ripf)

        def fl(t, _):
            rbase = sid * 112 + t * 16
            pltpu.sync_copy(acc_sh.at[pl.ds(rbase, 16)], hbuf)

            def fr(j, _):
                wv = ripf[t * 16 + j, :]
                for h in range(NH):
                    wh = _bcast16(wv, h)
                    for q in range(5):
                        col = (h * 5 + q) * 16
                        hbuf[j, pl.ds(col, 16)] = \
                            hbuf[j, pl.ds(col, 16)] * wh
                return 0
            lax.fori_loop(0, 16, fr, 0)
            pltpu.sync_copy(hbuf, out1_hbm.at[pl.ds(lo + rbase, 16)])
            return 0
        lax.fori_loop(0, 7, fl, 0)
        plsc.subcore_barrier()
        return 0
    lax.fori_loop(0, NBLK1 // 2, block_loop, 0)


def _k2c(src, dst, p1, h1p, rip1):
    k = pl.kernel(
        _k2c_body,
        out_type=jax.ShapeDtypeStruct((NT, HP), f32),
        mesh=_mesh(),
        compiler_params=pltpu.CompilerParams(**_SC_PARAMS),
        scratch_types=[
            pltpu.VMEM((SCCH,), i32),
            pltpu.VMEM((SCCH,), i32),
            pltpu.VMEM((SELCAP,), i32),
            pltpu.VMEM((SELCAP,), i32),
            pltpu.VMEM((SELCAP,), i32),
            pltpu.VMEM((16,), i32),
            pltpu.VMEM((16, 16), f32),
            pltpu.VMEM((16, HP), f32),
            pltpu.VMEM((16,), f32),
            pltpu.VMEM((112, 16), f32),
            pltpu.VMEM((6, HP), f32),
            pltpu.VMEM_SHARED((ACC1, HP), f32),
            pltpu.SemaphoreType.DMA,
            pltpu.SemaphoreType.DMA,
        ],
    )
    return k(src, dst, p1, h1p, rip1)


def _k4c_body(src_hbm, dst_hbm, p2_hbm, h2_hbm, rip2_hbm, out2_hbm,
              dstc, srcc, p2c, seld, sels, selp, gidx, hbuf, wscr,
              ripf, zbuf, acc_sh, sem, sem2):
    cid = lax.axis_index("c")
    sid = lax.axis_index("s")
    lanes = lax.iota(i32, 16)

    def zf(i, _):
        for j in range(OUT // 16):
            zbuf[i, pl.ds(j * 16, 16)] = _Z16F()
        return 0
    lax.fori_loop(0, 6, zf, 0)

    def sz(i, _):
        seld[pl.ds(i * 16, 16)] = _Z16I()
        sels[pl.ds(i * 16, 16)] = _Z16I()
        selp[pl.ds(i * 16, 16)] = _Z16F()
        return 0
    lax.fori_loop(0, SELCAP // 16, sz, 0)

    scan_base = sid * CH_B

    def process_group(goff, nval):
        dl = seld[pl.ds(goff, 16)]
        sl = sels[pl.ds(goff, 16)]
        pv = selp[pl.ds(goff, 16)]
        gidx[...] = jnp.where(lanes < nval, dl, jnp.full((16,), CB2, i32))
        pltpu.async_copy(h2_hbm.at[sl], hbuf, sem2).wait()

        def pe(j, _):
            wj = _bcast16(pv, j)
            for q in range(OUT // 16):
                col = q * 16
                hbuf[j, pl.ds(col, 16)] = hbuf[j, pl.ds(col, 16)] * wj
            return 0
        lax.fori_loop(0, nval, pe, 0)
        pltpu.sync_copy(hbuf, acc_sh.at[gidx], add=True)

    def block_loop(blk2, _):
        blk = 2 * blk2 + cid
        lo = blk * CB2
        hi = jnp.minimum(lo + CB2, jnp.int32(N))

        def zd(i, _):
            pltpu.sync_copy(zbuf, acc_sh.at[pl.ds(sid * 786 + i * 6, 6)])
            return 0
        lax.fori_loop(0, 131, zd, 0)
        plsc.subcore_barrier()

        def chunk_loop(c, cnt):
            cb = scan_base + c * SCCH
            pltpu.sync_copy(dst_hbm.at[pl.ds(cb, SCCH)], dstc)
            pltpu.sync_copy(src_hbm.at[pl.ds(cb, SCCH)], srcc)
            pltpu.sync_copy(p2_hbm.at[pl.ds(cb, SCCH)], p2c)

            def vec_loop(v, cnt):
                dv = dstc[pl.ds(v * 16, 16)]
                sv = srcc[pl.ds(v * 16, 16)]
                pw = p2c[pl.ds(v * 16, 16)]
                m = (dv >= lo) & (dv < hi)
                cs = plsc.cumsum(_ones_zeros(m))
                pos = jnp.where(m, cnt + cs - 1,
                                jnp.full((16,), 528, i32) + lanes)
                plsc.store_scatter(seld, [pos], dv - lo)
                plsc.store_scatter(sels, [pos], sv)
                plsc.store_scatter(selp, [pos], pw)
                return cnt + cs[15]
            cnt = lax.fori_loop(0, 32, vec_loop, cnt)
            ng = cnt // 16

            def grp(g, _):
                process_group(g * 16, jnp.int32(16))
                return 0
            lax.fori_loop(0, ng, grp, 0)
            rem = cnt - ng * 16
            td = seld[pl.ds(ng * 16, 16)]
            ts = sels[pl.ds(ng * 16, 16)]
            tp = selp[pl.ds(ng * 16, 16)]
            seld[pl.ds(0, 16)] = td
            sels[pl.ds(0, 16)] = ts
            selp[pl.ds(0, 16)] = tp
            return rem
        cnt_end = lax.fori_loop(0, NCH_B, chunk_loop, jnp.int32(0))

        @pl.when(cnt_end > 0)
        def _():
            process_group(0, cnt_end)
        plsc.subcore_barrier()

        pltpu.sync_copy(rip2_hbm.at[pl.ds(lo + sid * 784, 784)], ripf)

        def fl(t, _):
            rbase = sid * 784 + t * 16
            pltpu.sync_copy(acc_sh.at[pl.ds(rbase, 16)], hbuf)
            rw = ripf[pl.ds(t * 16, 16)]

            def fr(j, _):
                wj = _bcast16(rw, j)
                for q in range(OUT // 16):
                    col = q * 16
                    hbuf[j, pl.ds(col, 16)] = hbuf[j, pl.ds(col, 16)] * wj
                return 0
            lax.fori_loop(0, 16, fr, 0)
            pltpu.sync_copy(hbuf, out2_hbm.at[pl.ds(lo + rbase, 16)])
            return 0
        lax.fori_loop(0, 49, fl, 0)
        plsc.subcore_barrier()
        return 0
    lax.fori_loop(0, 2, block_loop, 0)


def _k4c(src, dst, p2, h2, rip2):
    k = pl.kernel(
        _k4c_body,
        out_type=jax.ShapeDtypeStruct((NT, OUT), f32),
        mesh=_mesh(),
        compiler_params=pltpu.CompilerParams(**_SC_PARAMS),
        scratch_types=[
            pltpu.VMEM((SCCH,), i32),
            pltpu.VMEM((SCCH,), i32),
            pltpu.VMEM((SCCH,), f32),
            pltpu.VMEM((SELCAP,), i32),
            pltpu.VMEM((SELCAP,), i32),
            pltpu.VMEM((SELCAP,), f32),
            pltpu.VMEM((16,), i32),
            pltpu.VMEM((16, OUT), f32),
            pltpu.VMEM((16,), f32),
            pltpu.VMEM((784,), f32),
            pltpu.VMEM((6, OUT), f32),
            pltpu.VMEM_SHARED((ACC2, OUT), f32),
            pltpu.SemaphoreType.DMA,
            pltpu.SemaphoreType.DMA,
        ],
    )
    return k(src, dst, p2, h2, rip2)


def _k5_body(out2_hbm, b2_hbm, batch_hbm, parts_hbm,
             rows, btl, pool, b2l, sem):
    cid = lax.axis_index("c")
    sid = lax.axis_index("s")
    wid = sid * 2 + cid
    base = wid * 1568
    pltpu.sync_copy(batch_hbm.at[pl.ds(base, 1568)], btl.at[pl.ds(0, 1568)])
    pltpu.sync_copy(b2_hbm, b2l)

    def zf(i, _):
        for q in range(OUT // 16):
            pool[i, pl.ds(q * 16, 16)] = _Z16F()
        return 0
    lax.fori_loop(0, 256, zf, 0)

    def chunk(c, _):
        pltpu.sync_copy(out2_hbm.at[pl.ds(base + c * 56, 56)], rows)

        def row(r, _):
            b = btl[pl.ds(c * 56 + r, 16)][0]
            for q in range(OUT // 16):
                col = q * 16
                v = jnp.maximum(
                    rows[r, pl.ds(col, 16)] + b2l[pl.ds(col, 16)], 0.0)
                pool[b, pl.ds(col, 16)] = jnp.maximum(
                    pool[b, pl.ds(col, 16)], v)
            return 0
        lax.fori_loop(0, 56, row, 0)
        return 0
    lax.fori_loop(0, 28, chunk, 0)
    pltpu.sync_copy(pool, parts_hbm.at[wid])


def _k5(out2, b2, batchp):
    k = pl.kernel(
        _k5_body,
        out_type=jax.ShapeDtypeStruct((32, 256, OUT), f32),
        mesh=_mesh(),
        compiler_params=pltpu.CompilerParams(**_SC_PARAMS),
        scratch_types=[
            pltpu.VMEM((56, OUT), f32),
            pltpu.VMEM((1584,), i32),
            pltpu.VMEM((256, OUT), f32),
            pltpu.VMEM((OUT,), f32),
            pltpu.SemaphoreType.DMA,
        ],
    )
    return k(out2, b2, batchp)


def kernel(x, edge_index, batch, W1, a_src1, a_dst1, b1, W2, a_src2, a_dst2,
           b2, Wf, bf):
    loop = jnp.arange(N, dtype=i32)
    src = jnp.concatenate(
        [edge_index[0].astype(i32), loop, jnp.zeros((EP - E - N,), i32)])
    dst = jnp.concatenate(
        [edge_index[1].astype(i32), loop, jnp.full((EP - E - N,), N, i32)])

    xp = jnp.zeros((NT, 128), f32).at[:N, :78].set(x)
    w1p = jnp.pad(W1.reshape(78, NH, 78), ((0, 50), (0, 0), (0, 2))) \
        .reshape(128, HP)
    oh = (jnp.arange(NH)[:, None] == jnp.arange(16)[None, :]).astype(f32)
    asrc = (jnp.pad(a_src1, ((0, 0), (0, 2)))[:, :, None]
            * oh[:, None, :]).reshape(HP, 16)
    adst = (jnp.pad(a_dst1, ((0, 0), (0, 2)))[:, :, None]
            * oh[:, None, :]).reshape(HP, 16)

    h1p, as1, ad1 = _k1(xp, w1p, asrc, adst)
    p1, parts1 = _k2a(src, dst, as1, ad1)
    pr = parts1.reshape(2, NT * 16 // 128, 128)
    rip1 = _recip(pr[0], pr[1], 128).reshape(NT, 16)
    out1 = _k2c(src, dst, p1, h1p, rip1)

    w2p = jnp.pad(W2.reshape(NH, 78, OUT), ((0, 0), (0, 2), (0, 0))) \
        .reshape(HP, OUT)
    b1p = jnp.pad(b1.reshape(NH, 78), ((0, 0), (0, 2))).reshape(1, HP)
    h2, as2m, ad2m = _k3(out1, w2p, b1p, a_src2, a_dst2)
    as2 = as2m.reshape(NT)
    ad2 = ad2m.reshape(NT)

    p2, parts2 = _k4a(src, dst, as2, ad2)
    pr2 = parts2.reshape(2, NT // 128, 128)
    rip2 = _recip(pr2[0], pr2[1], 56).reshape(NT)
    out2 = _k4c(src, dst, p2, h2, rip2)

    batchp = jnp.concatenate(
        [batch.astype(i32), jnp.full((NT - N,), 255, i32)])
    partials = _k5(out2, b2, batchp)
    return _k6(partials, Wf, bf.reshape(1, OUT))
